# Initial kernel scaffold; baseline (speedup 1.0000x reference)
import jax
import jax.numpy as jnp
from jax.experimental import pallas as pl


def kernel(pred_logits, targets, indices_b, indices_q, empty_weight):
    raise NotImplementedError("write your pallas kernel here")


# speedup vs baseline: 2.2987x; 2.2987x over previous
import functools

import jax
import jax.numpy as jnp
from jax import lax
from jax.experimental import pallas as pl
from jax.experimental.pallas import tpu as pltpu
from jax.experimental.pallas import tpu_sc as plsc

_B, _Q, _C = 64, 100, 41
_R = _B * _Q
_NC, _NS = 2, 16
_NW = _NC * _NS
_RPW = _R // _NW
_NG = 13
_RPAD = _NG * 16
_N = 80
_LN2 = 0.6931471805599453

_mesh = plsc.VectorSubcoreMesh(
    core_axis_name="c", subcore_axis_name="s", num_cores=_NC, num_subcores=_NS
)


@functools.partial(
    pl.kernel,
    out_type=(
        jax.ShapeDtypeStruct((_NW * 16,), jnp.float32),
        jax.ShapeDtypeStruct((_NW * 16,), jnp.float32),
    ),
    mesh=_mesh,
    compiler_params=pltpu.CompilerParams(needs_layout_passes=False),
    scratch_types=[
        pltpu.VMEM((_RPAD * _C,), jnp.float32),
        pltpu.VMEM((_RPAD,), jnp.float32),
        pltpu.VMEM((48,), jnp.float32),
        pltpu.VMEM((_N,), jnp.int32),
        pltpu.VMEM((_N,), jnp.int32),
        pltpu.VMEM((_N,), jnp.int32),
        pltpu.VMEM((32,), jnp.float32),
    ],
)
def _sc_loss(x_hbm, b_hbm, q_hbm, t_hbm, ew_hbm, num_hbm, den_hbm,
             x_v, tc_v, ew_v, b_v, q_v, t_v, out_v):
    wid = lax.axis_index("s") * _NC + lax.axis_index("c")
    lo = wid * _RPW

    pltpu.sync_copy(x_hbm.at[pl.ds(lo * _C, _RPW * _C)], x_v.at[pl.ds(0, _RPW * _C)])
    pltpu.sync_copy(b_hbm, b_v)
    pltpu.sync_copy(q_hbm, q_v)
    pltpu.sync_copy(t_hbm, t_v)
    pltpu.sync_copy(ew_hbm, ew_v)

    lanes = lax.broadcasted_iota(jnp.int32, (16,), 0)

    fill = jnp.full((16,), float(_C - 1), jnp.float32)
    for g in range(_NG):
        tc_v[pl.ds(g * 16, 16)] = fill
    for v in range(_N // 16):
        bb = b_v[pl.ds(v * 16, 16)]
        qq = q_v[pl.ds(v * 16, 16)]
        tt = t_v[pl.ds(v * 16, 16)].astype(jnp.float32)
        rloc = bb * _Q + qq - lo
        inrange = (rloc >= 0) & (rloc < _RPW)
        trash = _RPW + (lanes & 7)
        for l in range(16):
            idx = jnp.where(inrange & (lanes == l), rloc, trash)
            plsc.store_scatter(tc_v, [idx], tt)

    def _group(g, carry):
        nacc, dacc = carry
        row = g * 16 + lanes
        xb = row * _C
        m = plsc.load_gather(x_v, [xb])
        for c in range(1, _C):
            m = jnp.maximum(m, plsc.load_gather(x_v, [xb + c]))
        s = jnp.exp(plsc.load_gather(x_v, [xb]) - m)
        for c in range(1, _C):
            s = s + jnp.exp(plsc.load_gather(x_v, [xb + c]) - m)
        bits = plsc.bitcast(s, jnp.int32)
        e = (bits >> 23) - 127
        f = plsc.bitcast((bits & 0x007FFFFF) | 0x3F800000, jnp.float32)
        z = (f - 1.0) / (f + 1.0)
        z2 = z * z
        p = z * (2.0 + z2 * (2.0 / 3.0 + z2 * (2.0 / 5.0 + z2 * (2.0 / 7.0 + z2 * (2.0 / 9.0)))))
        lse = m + e.astype(jnp.float32) * _LN2 + p
        tc = plsc.load_gather(tc_v, [row]).astype(jnp.int32)
        w = plsc.load_gather(ew_v, [tc])
        xtc = plsc.load_gather(x_v, [xb + tc])
        zero = jnp.zeros((16,), jnp.float32)
        valid = row < _RPW
        nacc = nacc + jnp.where(valid, w * (lse - xtc), zero)
        dacc = dacc + jnp.where(valid, w, zero)
        return nacc, dacc

    z16 = jnp.zeros((16,), jnp.float32)
    nacc, dacc = lax.fori_loop(0, _NG, _group, (z16, z16))
    out_v[pl.ds(0, 16)] = nacc
    out_v[pl.ds(16, 16)] = dacc
    pltpu.sync_copy(out_v.at[pl.ds(0, 16)], num_hbm.at[pl.ds(wid * 16, 16)])
    pltpu.sync_copy(out_v.at[pl.ds(16, 16)], den_hbm.at[pl.ds(wid * 16, 16)])


def _tc_finish_body(num_ref, den_ref, o_ref):
    o_ref[0, 0] = jnp.sum(num_ref[...]) / jnp.sum(den_ref[...])


_tc_finish = pl.pallas_call(
    _tc_finish_body,
    out_shape=jax.ShapeDtypeStruct((1, 1), jnp.float32),
    out_specs=pl.BlockSpec(memory_space=pltpu.SMEM),
)


def kernel(pred_logits, targets, indices_b, indices_q, empty_weight):
    x = pred_logits.astype(jnp.float32).reshape(-1)
    ew = jnp.pad(empty_weight.astype(jnp.float32), (0, 48 - _C))
    num, den = _sc_loss(x, indices_b, indices_q, targets, ew)
    loss = _tc_finish(num.reshape(4, 128), den.reshape(4, 128))
    return loss[0, 0]


# speedup vs baseline: 2.3346x; 1.0156x over previous
import functools

import jax
import jax.numpy as jnp
from jax import lax
from jax.experimental import pallas as pl
from jax.experimental.pallas import tpu as pltpu
from jax.experimental.pallas import tpu_sc as plsc

_B, _Q, _C = 64, 100, 41
_R = _B * _Q
_NC, _NS = 2, 16
_NW = _NC * _NS
_BPW = _B // _NW
_RPW = _R // _NW
_NG = 13
_RPAD = _NG * 16
_N = 80
_LN2 = 0.6931471805599453

_mesh = plsc.VectorSubcoreMesh(
    core_axis_name="c", subcore_axis_name="s", num_cores=_NC, num_subcores=_NS
)


@functools.partial(
    pl.kernel,
    out_type=(
        jax.ShapeDtypeStruct((_NW * 16,), jnp.float32),
        jax.ShapeDtypeStruct((_NW * 16,), jnp.float32),
    ),
    mesh=_mesh,
    compiler_params=pltpu.CompilerParams(
        needs_layout_passes=False, disable_bounds_checks=True
    ),
    scratch_types=[
        pltpu.VMEM((_RPAD * _C,), jnp.float32),
        pltpu.VMEM((_RPAD,), jnp.float32),
        pltpu.VMEM((_C,), jnp.float32),
        pltpu.VMEM((_N,), jnp.int32),
        pltpu.VMEM((_N,), jnp.int32),
        pltpu.VMEM((_N,), jnp.int32),
        pltpu.VMEM((32,), jnp.float32),
    ],
)
def _sc_loss(x_hbm, b_hbm, q_hbm, t_hbm, ew_hbm, num_hbm, den_hbm,
             x_v, tc_v, ew_v, b_v, q_v, t_v, out_v):
    wid = lax.axis_index("s") * _NC + lax.axis_index("c")
    lo = wid * _RPW

    pltpu.sync_copy(x_hbm.at[pl.ds(lo * _C, _RPW * _C)], x_v.at[pl.ds(0, _RPW * _C)])
    pltpu.sync_copy(b_hbm, b_v)
    pltpu.sync_copy(q_hbm, q_v)
    pltpu.sync_copy(t_hbm, t_v)
    pltpu.sync_copy(ew_hbm, ew_v)

    lanes = lax.broadcasted_iota(jnp.int32, (16,), 0)

    fill = jnp.full((16,), float(_C - 1), jnp.float32)
    for g in range(_NG):
        tc_v[pl.ds(g * 16, 16)] = fill
    for v in range(_N // 16):
        bb = b_v[pl.ds(v * 16, 16)]
        qq = q_v[pl.ds(v * 16, 16)]
        tt = t_v[pl.ds(v * 16, 16)].astype(jnp.float32)
        rloc = bb * _Q + qq - lo
        inrange = (rloc >= 0) & (rloc < _RPW)
        trash = _RPW + (lanes & 7)
        for l in range(16):
            idx = jnp.where(inrange & (lanes == l), rloc, trash)
            plsc.store_scatter(tc_v, [idx], tt)

    def _gx(flat):
        return plsc.load_gather(x_v, [flat])

    def _group(g, carry):
        nacc, dacc = carry
        row = g * 16 + lanes
        xb = jnp.minimum(row, _RPW - 1) * _C
        m = _gx(xb)
        for c in range(1, _C):
            m = jnp.maximum(m, _gx(xb + c))
        s = jnp.exp(_gx(xb) - m)
        for c in range(1, _C):
            s = s + jnp.exp(_gx(xb + c) - m)
        bits = plsc.bitcast(s, jnp.int32)
        e = (bits >> 23) - 127
        f = plsc.bitcast((bits & 0x007FFFFF) | 0x3F800000, jnp.float32)
        z = (f - 1.0) / (f + 1.0)
        z2 = z * z
        p = z * (2.0 + z2 * (2.0 / 3.0 + z2 * (2.0 / 5.0 + z2 * (2.0 / 7.0 + z2 * (2.0 / 9.0)))))
        lse = m + e.astype(jnp.float32) * _LN2 + p
        tc = plsc.load_gather(tc_v, [row]).astype(jnp.int32)
        w = plsc.load_gather(ew_v, [tc])
        xtc = _gx(xb + tc)
        zero = jnp.zeros((16,), jnp.float32)
        valid = row < _RPW
        nacc = nacc + jnp.where(valid, w * (lse - xtc), zero)
        dacc = dacc + jnp.where(valid, w, zero)
        return nacc, dacc

    z16 = jnp.zeros((16,), jnp.float32)
    nacc, dacc = lax.fori_loop(0, _NG, _group, (z16, z16))
    out_v[pl.ds(0, 16)] = nacc
    out_v[pl.ds(16, 16)] = dacc
    pltpu.sync_copy(out_v.at[pl.ds(0, 16)], num_hbm.at[pl.ds(wid * 16, 16)])
    pltpu.sync_copy(out_v.at[pl.ds(16, 16)], den_hbm.at[pl.ds(wid * 16, 16)])


def _tc_finish_body(num_ref, den_ref, o_ref):
    o_ref[0, 0] = jnp.sum(num_ref[...]) / jnp.sum(den_ref[...])


_tc_finish = pl.pallas_call(
    _tc_finish_body,
    out_shape=jax.ShapeDtypeStruct((1, 1), jnp.float32),
    out_specs=pl.BlockSpec(memory_space=pltpu.SMEM),
)


def kernel(pred_logits, targets, indices_b, indices_q, empty_weight):
    x = pred_logits.reshape(-1)
    num, den = _sc_loss(x, indices_b, indices_q, targets, empty_weight)
    loss = _tc_finish(num.reshape(4, 128), den.reshape(4, 128))
    return loss[0, 0]
